# Initial kernel scaffold; baseline (speedup 1.0000x reference)
import jax
import jax.numpy as jnp
from jax.experimental import pallas as pl


def kernel(x, edge_index, batch, Wl1, bl1, Wr1, Wl2, bl2, Wr2, Wout, bout):
    raise NotImplementedError("write your pallas kernel here")


# speedup vs baseline: 4.8986x; 4.8986x over previous
import functools

import jax
import jax.numpy as jnp
from jax import lax
from jax.experimental import pallas as pl
from jax.experimental.pallas import tpu as pltpu
from jax.experimental.pallas import tpu_sc as plsc

N = 10000
E = 320000
D = 128
G = 64

NC = 2
NS = 16
NW = NC * NS
EPW = E // NW
C = 80
NCHUNK = EPW // C
RPT = 624
RTAIL = N - NS * RPT


def _sc_agg_body(with_deg, h_hbm, src_hbm, dst_hbm, *refs):
    if with_deg:
        (acc_out, deg_out, acc_sh, sidx_v, didx_v, rows_v, ones_v,
         zbuf, sem) = refs
    else:
        acc_out, acc_sh, sidx_v, didx_v, rows_v, zbuf, sem = refs

    cid = lax.axis_index("c")
    sid = lax.axis_index("s")
    wid = sid * NC + cid
    base = wid * EPW

    def zero_my_acc_slice():
        def za(j, _):
            pltpu.sync_copy(zbuf,
                            acc_sh.at[pl.ds(sid * RPT + j * 16, 16)])
            return _

        lax.fori_loop(0, RPT // 16, za, 0)

        @pl.when(sid == NS - 1)
        def _():
            pltpu.sync_copy(zbuf.at[pl.ds(0, RTAIL)],
                            acc_sh.at[pl.ds(NS * RPT, RTAIL)])

    def copy_my_acc_slice(out_ref):
        pltpu.sync_copy(acc_sh.at[pl.ds(sid * RPT, RPT)],
                        out_ref.at[cid, pl.ds(sid * RPT, RPT)])

        @pl.when(sid == NS - 1)
        def _():
            pltpu.sync_copy(acc_sh.at[pl.ds(NS * RPT, RTAIL)],
                            out_ref.at[cid, pl.ds(NS * RPT, RTAIL)])

    zeros16 = jnp.zeros((16,), jnp.float32)

    def zb(i, _):
        zbuf[i // (D // 16), pl.ds((i % (D // 16)) * 16, 16)] = zeros16
        return _

    lax.fori_loop(0, 16 * (D // 16), zb, 0)

    zero_my_acc_slice()

    if with_deg:
        ones16 = jnp.ones((16,), jnp.float32)

        def ob(i, _):
            ones_v[i // (D // 16), pl.ds((i % (D // 16)) * 16, 16)] = ones16
            return _

        lax.fori_loop(0, C * (D // 16), ob, 0)

        plsc.subcore_barrier()

        def dchunk(g, _):
            pltpu.sync_copy(dst_hbm.at[pl.ds(base + g * C, C)], didx_v)
            pltpu.sync_copy(ones_v, acc_sh.at[didx_v], add=True)
            return _

        lax.fori_loop(0, NCHUNK, dchunk, 0)

        plsc.subcore_barrier()
        copy_my_acc_slice(deg_out)
        zero_my_acc_slice()

    plsc.subcore_barrier()

    def chunk(g, _):
        off = base + g * C
        pltpu.sync_copy(src_hbm.at[pl.ds(off, C)], sidx_v)
        pltpu.sync_copy(dst_hbm.at[pl.ds(off, C)], didx_v)
        pltpu.async_copy(h_hbm.at[sidx_v], rows_v, sem).wait()
        pltpu.sync_copy(rows_v, acc_sh.at[didx_v], add=True)
        return _

    lax.fori_loop(0, NCHUNK, chunk, 0)

    plsc.subcore_barrier()
    copy_my_acc_slice(acc_out)


def _make_sc_agg(with_deg):
    mesh = plsc.VectorSubcoreMesh(core_axis_name="c", subcore_axis_name="s")
    out_type = [jax.ShapeDtypeStruct((NC, N, D), jnp.float32)]
    scratch = [
        pltpu.VMEM_SHARED((N, D), jnp.float32),
    ]
    if with_deg:
        out_type.append(jax.ShapeDtypeStruct((NC, N, D), jnp.float32))
    scratch += [
        pltpu.VMEM((C,), jnp.int32),
        pltpu.VMEM((C,), jnp.int32),
        pltpu.VMEM((C, D), jnp.float32),
    ]
    if with_deg:
        scratch.append(pltpu.VMEM((C, D), jnp.float32))
    scratch.append(pltpu.VMEM((16, D), jnp.float32))
    scratch.append(pltpu.SemaphoreType.DMA)

    return pl.kernel(
        functools.partial(_sc_agg_body, with_deg),
        out_type=tuple(out_type) if with_deg else out_type[0],
        mesh=mesh,
        scratch_types=scratch,
    )


_sc_agg_deg = _make_sc_agg(True)
_sc_agg = _make_sc_agg(False)


BN = 2000
BW = 16


def _tc_layer_body(x_ref, accp_ref, degp_ref, wl_ref, bl_ref, wr_ref, o_ref):
    acc = accp_ref[0] + accp_ref[1]
    dsum = degp_ref[0] + degp_ref[1]
    deg = jnp.maximum(dsum[:, 0:1], 1.0)
    mean = acc / deg
    h = (lax.dot_general(mean, wl_ref[...], (((1,), (1,)), ((), ())),
                         preferred_element_type=jnp.float32)
         + bl_ref[...]
         + lax.dot_general(x_ref[...], wr_ref[...], (((1,), (1,)), ((), ())),
                           preferred_element_type=jnp.float32))
    o_ref[...] = jnp.maximum(h, 0.0)


def _tc_layer(x, accp, degp, wl, bl, wr):
    grid = (N // BN,)
    return pl.pallas_call(
        _tc_layer_body,
        grid=grid,
        in_specs=[
            pl.BlockSpec((BN, D), lambda i: (i, 0)),
            pl.BlockSpec((NC, BN, D), lambda i: (0, i, 0)),
            pl.BlockSpec((NC, BN, D), lambda i: (0, i, 0)),
            pl.BlockSpec((D, D), lambda i: (0, 0)),
            pl.BlockSpec((1, D), lambda i: (0, 0)),
            pl.BlockSpec((D, D), lambda i: (0, 0)),
        ],
        out_specs=pl.BlockSpec((BN, D), lambda i: (i, 0)),
        out_shape=jax.ShapeDtypeStruct((N, D), jnp.float32),
    )(x, accp, degp, wl, bl, wr)


def _tc_final_body(h1_ref, accp_ref, degp_ref, batch_ref, wl_ref, bl_ref,
                   wr_ref, wout_ref, bout_ref, o_ref, ps_ref, pc_ref):
    i = pl.program_id(0)
    acc = accp_ref[0] + accp_ref[1]
    dsum = degp_ref[0] + degp_ref[1]
    deg = jnp.maximum(dsum[:, 0:1], 1.0)
    mean = acc / deg
    h = (lax.dot_general(mean, wl_ref[...], (((1,), (1,)), ((), ())),
                         preferred_element_type=jnp.float32)
         + bl_ref[...]
         + lax.dot_general(h1_ref[...], wr_ref[...], (((1,), (1,)), ((), ())),
                           preferred_element_type=jnp.float32))
    h2 = jnp.maximum(h, 0.0)

    bvals = batch_ref[:, 0:1]
    gids = lax.broadcasted_iota(jnp.int32, (BN, G), 1)
    oh = (bvals == gids).astype(jnp.float32)

    ps = lax.dot_general(oh, h2, (((0,), (0,)), ((), ())),
                         preferred_element_type=jnp.float32)
    ones_b = jnp.ones((BN, D), jnp.float32)
    pc = lax.dot_general(oh, ones_b, (((0,), (0,)), ((), ())),
                         preferred_element_type=jnp.float32)

    @pl.when(i == 0)
    def _():
        ps_ref[...] = ps
        pc_ref[...] = pc

    @pl.when(i != 0)
    def _():
        ps_ref[...] = ps_ref[...] + ps
        pc_ref[...] = pc_ref[...] + pc

    @pl.when(i == pl.num_programs(0) - 1)
    def _():
        pooled = ps_ref[...] / jnp.maximum(pc_ref[...], 1.0)
        o_ref[...] = (
            lax.dot_general(pooled, wout_ref[...], (((1,), (1,)), ((), ())),
                            preferred_element_type=jnp.float32)
            + bout_ref[...])


def _tc_final(h1, accp, degp, batchb, wl, bl, wr, wout, bout):
    grid = (N // BN,)
    return pl.pallas_call(
        _tc_final_body,
        grid=grid,
        in_specs=[
            pl.BlockSpec((BN, D), lambda i: (i, 0)),
            pl.BlockSpec((NC, BN, D), lambda i: (0, i, 0)),
            pl.BlockSpec((NC, BN, D), lambda i: (0, i, 0)),
            pl.BlockSpec((BN, BW), lambda i: (i, 0)),
            pl.BlockSpec((D, D), lambda i: (0, 0)),
            pl.BlockSpec((1, D), lambda i: (0, 0)),
            pl.BlockSpec((D, D), lambda i: (0, 0)),
            pl.BlockSpec((D, D), lambda i: (0, 0)),
            pl.BlockSpec((1, D), lambda i: (0, 0)),
        ],
        out_specs=pl.BlockSpec((G, D), lambda i: (0, 0)),
        out_shape=jax.ShapeDtypeStruct((G, D), jnp.float32),
        scratch_shapes=[
            pltpu.VMEM((G, D), jnp.float32),
            pltpu.VMEM((G, D), jnp.float32),
        ],
    )(h1, accp, degp, batchb, wl, bl, wr, wout, bout)


def kernel(x, edge_index, batch, Wl1, bl1, Wr1, Wl2, bl2, Wr2, Wout, bout):
    src = edge_index[0].astype(jnp.int32)
    dst = edge_index[1].astype(jnp.int32)
    batchb = jnp.broadcast_to(
        batch.astype(jnp.int32)[:, None], (N, BW))

    accp1, degp = _sc_agg_deg(x, src, dst)
    h1 = _tc_layer(x, accp1, degp, Wl1, bl1.reshape(1, D), Wr1)
    accp2 = _sc_agg(h1, src, dst)
    out = _tc_final(h1, accp2, degp, batchb, Wl2, bl2.reshape(1, D), Wr2,
                    Wout, bout.reshape(1, D))
    return out


# speedup vs baseline: 9.4282x; 1.9247x over previous
import functools

import jax
import jax.numpy as jnp
from jax import lax
from jax.experimental import pallas as pl
from jax.experimental.pallas import tpu as pltpu
from jax.experimental.pallas import tpu_sc as plsc

N = 10000
E = 320000
D = 128
G = 64

NC = 2
NS = 16
NW = NC * NS
EPW = E // NW
C = 80
NCHUNK = EPW // C
NBUF = 4
RPT = 624
RTAIL = N - NS * RPT


def _sc_agg_body(with_deg, h_hbm, src_hbm, dst_hbm, *refs):
    if with_deg:
        (acc_out, deg_out, acc_sh, sidx_v, didx_v, rows_v, zbuf,
         isem_s, isem_d, gsem) = refs
    else:
        (acc_out, acc_sh, sidx_v, didx_v, rows_v, zbuf,
         isem_s, isem_d, gsem) = refs

    cid = lax.axis_index("c")
    sid = lax.axis_index("s")
    wid = sid * NC + cid
    base = wid * EPW

    def zero_my_acc_slice():
        def za(j, _):
            pltpu.sync_copy(zbuf,
                            acc_sh.at[pl.ds(sid * RPT + j * 16, 16)])
            return _

        lax.fori_loop(0, RPT // 16, za, 0)

        @pl.when(sid == NS - 1)
        def _():
            pltpu.sync_copy(zbuf.at[pl.ds(0, RTAIL)],
                            acc_sh.at[pl.ds(NS * RPT, RTAIL)])

    def copy_my_acc_slice(out_ref):
        pltpu.sync_copy(acc_sh.at[pl.ds(sid * RPT, RPT)],
                        out_ref.at[cid, pl.ds(sid * RPT, RPT)])

        @pl.when(sid == NS - 1)
        def _():
            pltpu.sync_copy(acc_sh.at[pl.ds(NS * RPT, RTAIL)],
                            out_ref.at[cid, pl.ds(NS * RPT, RTAIL)])

    zeros16 = jnp.zeros((16,), jnp.float32)

    def zb(i, _):
        zbuf[i // (D // 16), pl.ds((i % (D // 16)) * 16, 16)] = zeros16
        return _

    lax.fori_loop(0, 16 * (D // 16), zb, 0)

    zero_my_acc_slice()

    def start_didx(g, b):
        pltpu.async_copy(dst_hbm.at[pl.ds(base + g * C, C)], didx_v[b],
                         isem_d[b])

    def wait_didx(g, b):
        pltpu.make_async_copy(dst_hbm.at[pl.ds(base + g * C, C)], didx_v[b],
                              isem_d[b]).wait()

    def start_sidx(g, b):
        pltpu.async_copy(src_hbm.at[pl.ds(base + g * C, C)], sidx_v[b],
                         isem_s[b])

    def wait_sidx(g, b):
        pltpu.make_async_copy(src_hbm.at[pl.ds(base + g * C, C)], sidx_v[b],
                              isem_s[b]).wait()

    if with_deg:
        ones16 = jnp.ones((16,), jnp.float32)

        def ob(i, _):
            rows_v[0][i // (D // 16), pl.ds((i % (D // 16)) * 16, 16)] = ones16
            return _

        lax.fori_loop(0, C * (D // 16), ob, 0)

        plsc.subcore_barrier()

        start_didx(0, 0)
        start_didx(1, 1)

        def dpair(i, carry):
            for b in range(2):
                g = i * 2 + b
                wait_didx(g, b)
                pltpu.sync_copy(rows_v[0], acc_sh.at[didx_v[b]], add=True)

                @pl.when(g + 2 < NCHUNK)
                def _(b=b, g=g):
                    start_didx(g + 2, b)
            return carry

        lax.fori_loop(0, NCHUNK // 2, dpair, 0)

        wait_didx(NCHUNK - 1, 0)
        pltpu.sync_copy(rows_v[0], acc_sh.at[didx_v[0]], add=True)

        plsc.subcore_barrier()
        copy_my_acc_slice(deg_out)
        zero_my_acc_slice()

    plsc.subcore_barrier()

    for b in range(NBUF):
        start_sidx(b, b)
        start_didx(b, b)

    def mquad(i, carry):
        q = i * NBUF
        descs = []
        for b in range(2):
            wait_sidx(q + b, b)
            wait_didx(q + b, b)
            descs.append(
                pltpu.async_copy(h_hbm.at[sidx_v[b]], rows_v[b], gsem[b]))
        for b in range(NBUF):
            g = q + b
            nb = b + 2
            if nb < NBUF:
                wait_sidx(q + nb, nb)
                wait_didx(q + nb, nb)
                descs.append(
                    pltpu.async_copy(h_hbm.at[sidx_v[nb]], rows_v[nb],
                                     gsem[nb]))
            descs[b].wait()
            pltpu.sync_copy(rows_v[b], acc_sh.at[didx_v[b]], add=True)

            @pl.when(g + NBUF < NCHUNK)
            def _(b=b, g=g):
                start_sidx(g + NBUF, b)
                start_didx(g + NBUF, b)
        return carry

    lax.fori_loop(0, (NCHUNK - 1) // NBUF, mquad, 0)

    gl = NCHUNK - 1
    bl = gl % NBUF
    wait_sidx(gl, bl)
    wait_didx(gl, bl)
    pltpu.async_copy(h_hbm.at[sidx_v[bl]], rows_v[bl], gsem[bl]).wait()
    pltpu.sync_copy(rows_v[bl], acc_sh.at[didx_v[bl]], add=True)

    plsc.subcore_barrier()
    copy_my_acc_slice(acc_out)


def _make_sc_agg(with_deg):
    mesh = plsc.VectorSubcoreMesh(core_axis_name="c", subcore_axis_name="s")
    out_type = [jax.ShapeDtypeStruct((NC, N, D), jnp.float32)]
    scratch = [
        pltpu.VMEM_SHARED((N, D), jnp.float32),
    ]
    if with_deg:
        out_type.append(jax.ShapeDtypeStruct((NC, N, D), jnp.float32))
    scratch += [
        [pltpu.VMEM((C,), jnp.int32) for _ in range(NBUF)],
        [pltpu.VMEM((C,), jnp.int32) for _ in range(NBUF)],
        [pltpu.VMEM((C, D), jnp.float32) for _ in range(NBUF)],
        pltpu.VMEM((16, D), jnp.float32),
        [pltpu.SemaphoreType.DMA for _ in range(NBUF)],
        [pltpu.SemaphoreType.DMA for _ in range(NBUF)],
        [pltpu.SemaphoreType.DMA for _ in range(NBUF)],
    ]

    return pl.kernel(
        functools.partial(_sc_agg_body, with_deg),
        out_type=tuple(out_type) if with_deg else out_type[0],
        mesh=mesh,
        scratch_types=scratch,
    )


_sc_agg_deg = _make_sc_agg(True)
_sc_agg = _make_sc_agg(False)


BN = 2000
BW = 16


def _tc_layer_body(x_ref, accp_ref, degp_ref, wl_ref, bl_ref, wr_ref, o_ref):
    acc = accp_ref[0] + accp_ref[1]
    dsum = degp_ref[0] + degp_ref[1]
    deg = jnp.maximum(dsum[:, 0:1], 1.0)
    mean = acc / deg
    h = (lax.dot_general(mean, wl_ref[...], (((1,), (1,)), ((), ())),
                         preferred_element_type=jnp.float32)
         + bl_ref[...]
         + lax.dot_general(x_ref[...], wr_ref[...], (((1,), (1,)), ((), ())),
                           preferred_element_type=jnp.float32))
    o_ref[...] = jnp.maximum(h, 0.0)


def _tc_layer(x, accp, degp, wl, bl, wr):
    grid = (N // BN,)
    return pl.pallas_call(
        _tc_layer_body,
        grid=grid,
        in_specs=[
            pl.BlockSpec((BN, D), lambda i: (i, 0)),
            pl.BlockSpec((NC, BN, D), lambda i: (0, i, 0)),
            pl.BlockSpec((NC, BN, D), lambda i: (0, i, 0)),
            pl.BlockSpec((D, D), lambda i: (0, 0)),
            pl.BlockSpec((1, D), lambda i: (0, 0)),
            pl.BlockSpec((D, D), lambda i: (0, 0)),
        ],
        out_specs=pl.BlockSpec((BN, D), lambda i: (i, 0)),
        out_shape=jax.ShapeDtypeStruct((N, D), jnp.float32),
    )(x, accp, degp, wl, bl, wr)


def _tc_final_body(h1_ref, accp_ref, degp_ref, batch_ref, wl_ref, bl_ref,
                   wr_ref, wout_ref, bout_ref, o_ref, ps_ref, pc_ref):
    i = pl.program_id(0)
    acc = accp_ref[0] + accp_ref[1]
    dsum = degp_ref[0] + degp_ref[1]
    deg = jnp.maximum(dsum[:, 0:1], 1.0)
    mean = acc / deg
    h = (lax.dot_general(mean, wl_ref[...], (((1,), (1,)), ((), ())),
                         preferred_element_type=jnp.float32)
         + bl_ref[...]
         + lax.dot_general(h1_ref[...], wr_ref[...], (((1,), (1,)), ((), ())),
                           preferred_element_type=jnp.float32))
    h2 = jnp.maximum(h, 0.0)

    bvals = batch_ref[:, 0:1]
    gids = lax.broadcasted_iota(jnp.int32, (BN, G), 1)
    oh = (bvals == gids).astype(jnp.float32)

    ps = lax.dot_general(oh, h2, (((0,), (0,)), ((), ())),
                         preferred_element_type=jnp.float32)
    ones_b = jnp.ones((BN, D), jnp.float32)
    pc = lax.dot_general(oh, ones_b, (((0,), (0,)), ((), ())),
                         preferred_element_type=jnp.float32)

    @pl.when(i == 0)
    def _():
        ps_ref[...] = ps
        pc_ref[...] = pc

    @pl.when(i != 0)
    def _():
        ps_ref[...] = ps_ref[...] + ps
        pc_ref[...] = pc_ref[...] + pc

    @pl.when(i == pl.num_programs(0) - 1)
    def _():
        pooled = ps_ref[...] / jnp.maximum(pc_ref[...], 1.0)
        o_ref[...] = (
            lax.dot_general(pooled, wout_ref[...], (((1,), (1,)), ((), ())),
                            preferred_element_type=jnp.float32)
            + bout_ref[...])


def _tc_final(h1, accp, degp, batchb, wl, bl, wr, wout, bout):
    grid = (N // BN,)
    return pl.pallas_call(
        _tc_final_body,
        grid=grid,
        in_specs=[
            pl.BlockSpec((BN, D), lambda i: (i, 0)),
            pl.BlockSpec((NC, BN, D), lambda i: (0, i, 0)),
            pl.BlockSpec((NC, BN, D), lambda i: (0, i, 0)),
            pl.BlockSpec((BN, BW), lambda i: (i, 0)),
            pl.BlockSpec((D, D), lambda i: (0, 0)),
            pl.BlockSpec((1, D), lambda i: (0, 0)),
            pl.BlockSpec((D, D), lambda i: (0, 0)),
            pl.BlockSpec((D, D), lambda i: (0, 0)),
            pl.BlockSpec((1, D), lambda i: (0, 0)),
        ],
        out_specs=pl.BlockSpec((G, D), lambda i: (0, 0)),
        out_shape=jax.ShapeDtypeStruct((G, D), jnp.float32),
        scratch_shapes=[
            pltpu.VMEM((G, D), jnp.float32),
            pltpu.VMEM((G, D), jnp.float32),
        ],
    )(h1, accp, degp, batchb, wl, bl, wr, wout, bout)


def kernel(x, edge_index, batch, Wl1, bl1, Wr1, Wl2, bl2, Wr2, Wout, bout):
    src = edge_index[0].astype(jnp.int32)
    dst = edge_index[1].astype(jnp.int32)
    batchb = jnp.broadcast_to(
        batch.astype(jnp.int32)[:, None], (N, BW))

    accp1, degp = _sc_agg_deg(x, src, dst)
    h1 = _tc_layer(x, accp1, degp, Wl1, bl1.reshape(1, D), Wr1)
    accp2 = _sc_agg(h1, src, dst)
    out = _tc_final(h1, accp2, degp, batchb, Wl2, bl2.reshape(1, D), Wr2,
                    Wout, bout.reshape(1, D))
    return out


# speedup vs baseline: 10.6251x; 1.1269x over previous
import functools

import jax
import jax.numpy as jnp
from jax import lax
from jax.experimental import pallas as pl
from jax.experimental.pallas import tpu as pltpu
from jax.experimental.pallas import tpu_sc as plsc

N = 10000
E = 320000
D = 128
G = 64

NC = 2
NS = 16
NW = NC * NS
EPW = E // NW
C = 80
NCHUNK = EPW // C
NBUF = 2
RPT = 624
RTAIL = N - NS * RPT


def _sc_agg_body(with_deg, h_hbm, src_hbm, dst_hbm, *refs):
    if with_deg:
        (acc_out, deg_out, acc_sh, sidx_v, didx_v, rows_v, zbuf,
         zsem, isem, gsem) = refs
    else:
        (acc_out, acc_sh, sidx_v, didx_v, rows_v, zbuf,
         zsem, isem, gsem) = refs

    cid = lax.axis_index("c")
    sid = lax.axis_index("s")
    wid = sid * NC + cid

    def zero_my_acc_slice():
        def za(j, _):
            pltpu.async_copy(zbuf,
                             acc_sh.at[pl.ds(sid * RPT + j * 16, 16)], zsem)
            return _

        lax.fori_loop(0, RPT // 16, za, 0)

        @pl.when(sid == NS - 1)
        def _():
            pltpu.async_copy(zbuf.at[pl.ds(0, RTAIL)],
                             acc_sh.at[pl.ds(NS * RPT, RTAIL)], zsem)

        def zw(j, _):
            pltpu.make_async_copy(
                zbuf, acc_sh.at[pl.ds(sid * RPT + j * 16, 16)], zsem).wait()
            return _

        lax.fori_loop(0, RPT // 16, zw, 0)

        @pl.when(sid == NS - 1)
        def _():
            pltpu.make_async_copy(
                zbuf.at[pl.ds(0, RTAIL)],
                acc_sh.at[pl.ds(NS * RPT, RTAIL)], zsem).wait()

    def copy_my_acc_slice(out_ref):
        pltpu.sync_copy(acc_sh.at[pl.ds(sid * RPT, RPT)],
                        out_ref.at[cid, pl.ds(sid * RPT, RPT)])

        @pl.when(sid == NS - 1)
        def _():
            pltpu.sync_copy(acc_sh.at[pl.ds(NS * RPT, RTAIL)],
                            out_ref.at[cid, pl.ds(NS * RPT, RTAIL)])

    zeros16 = jnp.zeros((16,), jnp.float32)

    def zb(i, _):
        zbuf[i // (D // 16), pl.ds((i % (D // 16)) * 16, 16)] = zeros16
        return _

    lax.fori_loop(0, 16 * (D // 16), zb, 0)

    zero_my_acc_slice()

    base = wid * EPW
    pltpu.sync_copy(dst_hbm.at[wid], didx_v)

    def start_sidx(g, b):
        pltpu.async_copy(src_hbm.at[pl.ds(base + g * C, C)], sidx_v[b],
                         isem[b])

    def wait_sidx(g, b):
        pltpu.make_async_copy(src_hbm.at[pl.ds(base + g * C, C)], sidx_v[b],
                              isem[b]).wait()

    def start_gather(g, b):
        return pltpu.async_copy(h_hbm.at[sidx_v[b]], rows_v[b], gsem[b])

    def wait_gather(g, b):
        pltpu.make_async_copy(h_hbm.at[sidx_v[b]], rows_v[b],
                              gsem[b]).wait()

    def scatter(g, b):
        pltpu.sync_copy(rows_v[b], acc_sh.at[didx_v.at[g]], add=True)

    if with_deg:
        ones16 = jnp.ones((16,), jnp.float32)

        def ob(i, _):
            rows_v[0][i // (D // 16), pl.ds((i % (D // 16)) * 16, 16)] = ones16
            return _

        lax.fori_loop(0, C * (D // 16), ob, 0)

        plsc.subcore_barrier()

        def dchunk(g, carry):
            pltpu.sync_copy(rows_v[0], acc_sh.at[didx_v.at[g]], add=True)
            return carry

        lax.fori_loop(0, NCHUNK, dchunk, 0)

        plsc.subcore_barrier()
        copy_my_acc_slice(deg_out)
        zero_my_acc_slice()

    plsc.subcore_barrier()

    start_sidx(0, 0)
    wait_sidx(0, 0)
    start_gather(0, 0)
    start_sidx(1, 1)

    def mpair(i, carry):
        g = i * 2
        wait_sidx(g + 1, 1)
        start_gather(g + 1, 1)
        wait_gather(g, 0)

        @pl.when(g + 2 < NCHUNK)
        def _():
            start_sidx(g + 2, 0)

        scatter(g, 0)

        @pl.when(g + 2 < NCHUNK)
        def _():
            wait_sidx(g + 2, 0)
            start_gather(g + 2, 0)

        wait_gather(g + 1, 1)

        @pl.when(g + 3 < NCHUNK)
        def _():
            start_sidx(g + 3, 1)

        scatter(g + 1, 1)
        return carry

    lax.fori_loop(0, NCHUNK // 2, mpair, 0)

    wait_gather(NCHUNK - 1, 0)
    scatter(NCHUNK - 1, 0)

    plsc.subcore_barrier()
    copy_my_acc_slice(acc_out)


def _make_sc_agg(with_deg):
    mesh = plsc.VectorSubcoreMesh(core_axis_name="c", subcore_axis_name="s")
    out_type = [jax.ShapeDtypeStruct((NC, N, D), jnp.float32)]
    scratch = [
        pltpu.VMEM_SHARED((N, D), jnp.float32),
    ]
    if with_deg:
        out_type.append(jax.ShapeDtypeStruct((NC, N, D), jnp.float32))
    scratch += [
        [pltpu.VMEM((C,), jnp.int32) for _ in range(NBUF)],
        pltpu.VMEM((NCHUNK, C), jnp.int32),
        [pltpu.VMEM((C, D), jnp.float32) for _ in range(NBUF)],
        pltpu.VMEM((16, D), jnp.float32),
        pltpu.SemaphoreType.DMA,
        [pltpu.SemaphoreType.DMA for _ in range(NBUF)],
        [pltpu.SemaphoreType.DMA for _ in range(NBUF)],
    ]

    return pl.kernel(
        functools.partial(_sc_agg_body, with_deg),
        out_type=tuple(out_type) if with_deg else out_type[0],
        mesh=mesh,
        scratch_types=scratch,
    )


_sc_agg_deg = _make_sc_agg(True)
_sc_agg = _make_sc_agg(False)


BN = 2000
BW = 16


def _tc_layer_body(x_ref, accp_ref, degp_ref, wl_ref, bl_ref, wr_ref, o_ref):
    acc = accp_ref[0] + accp_ref[1]
    dsum = degp_ref[0] + degp_ref[1]
    deg = jnp.maximum(dsum[:, 0:1], 1.0)
    mean = acc / deg
    h = (lax.dot_general(mean, wl_ref[...], (((1,), (1,)), ((), ())),
                         preferred_element_type=jnp.float32)
         + bl_ref[...]
         + lax.dot_general(x_ref[...], wr_ref[...], (((1,), (1,)), ((), ())),
                           preferred_element_type=jnp.float32))
    o_ref[...] = jnp.maximum(h, 0.0)


def _tc_layer(x, accp, degp, wl, bl, wr):
    grid = (N // BN,)
    return pl.pallas_call(
        _tc_layer_body,
        grid=grid,
        in_specs=[
            pl.BlockSpec((BN, D), lambda i: (i, 0)),
            pl.BlockSpec((NC, BN, D), lambda i: (0, i, 0)),
            pl.BlockSpec((NC, BN, D), lambda i: (0, i, 0)),
            pl.BlockSpec((D, D), lambda i: (0, 0)),
            pl.BlockSpec((1, D), lambda i: (0, 0)),
            pl.BlockSpec((D, D), lambda i: (0, 0)),
        ],
        out_specs=pl.BlockSpec((BN, D), lambda i: (i, 0)),
        out_shape=jax.ShapeDtypeStruct((N, D), jnp.float32),
    )(x, accp, degp, wl, bl, wr)


def _tc_final_body(h1_ref, accp_ref, degp_ref, batch_ref, wl_ref, bl_ref,
                   wr_ref, wout_ref, bout_ref, o_ref, ps_ref, pc_ref):
    i = pl.program_id(0)
    acc = accp_ref[0] + accp_ref[1]
    dsum = degp_ref[0] + degp_ref[1]
    deg = jnp.maximum(dsum[:, 0:1], 1.0)
    mean = acc / deg
    h = (lax.dot_general(mean, wl_ref[...], (((1,), (1,)), ((), ())),
                         preferred_element_type=jnp.float32)
         + bl_ref[...]
         + lax.dot_general(h1_ref[...], wr_ref[...], (((1,), (1,)), ((), ())),
                           preferred_element_type=jnp.float32))
    h2 = jnp.maximum(h, 0.0)

    bvals = batch_ref[:, 0:1]
    gids = lax.broadcasted_iota(jnp.int32, (BN, G), 1)
    oh = (bvals == gids).astype(jnp.float32)

    ps = lax.dot_general(oh, h2, (((0,), (0,)), ((), ())),
                         preferred_element_type=jnp.float32)
    ones_b = jnp.ones((BN, D), jnp.float32)
    pc = lax.dot_general(oh, ones_b, (((0,), (0,)), ((), ())),
                         preferred_element_type=jnp.float32)

    @pl.when(i == 0)
    def _():
        ps_ref[...] = ps
        pc_ref[...] = pc

    @pl.when(i != 0)
    def _():
        ps_ref[...] = ps_ref[...] + ps
        pc_ref[...] = pc_ref[...] + pc

    @pl.when(i == pl.num_programs(0) - 1)
    def _():
        pooled = ps_ref[...] / jnp.maximum(pc_ref[...], 1.0)
        o_ref[...] = (
            lax.dot_general(pooled, wout_ref[...], (((1,), (1,)), ((), ())),
                            preferred_element_type=jnp.float32)
            + bout_ref[...])


def _tc_final(h1, accp, degp, batchb, wl, bl, wr, wout, bout):
    grid = (N // BN,)
    return pl.pallas_call(
        _tc_final_body,
        grid=grid,
        in_specs=[
            pl.BlockSpec((BN, D), lambda i: (i, 0)),
            pl.BlockSpec((NC, BN, D), lambda i: (0, i, 0)),
            pl.BlockSpec((NC, BN, D), lambda i: (0, i, 0)),
            pl.BlockSpec((BN, BW), lambda i: (i, 0)),
            pl.BlockSpec((D, D), lambda i: (0, 0)),
            pl.BlockSpec((1, D), lambda i: (0, 0)),
            pl.BlockSpec((D, D), lambda i: (0, 0)),
            pl.BlockSpec((D, D), lambda i: (0, 0)),
            pl.BlockSpec((1, D), lambda i: (0, 0)),
        ],
        out_specs=pl.BlockSpec((G, D), lambda i: (0, 0)),
        out_shape=jax.ShapeDtypeStruct((G, D), jnp.float32),
        scratch_shapes=[
            pltpu.VMEM((G, D), jnp.float32),
            pltpu.VMEM((G, D), jnp.float32),
        ],
    )(h1, accp, degp, batchb, wl, bl, wr, wout, bout)


def kernel(x, edge_index, batch, Wl1, bl1, Wr1, Wl2, bl2, Wr2, Wout, bout):
    src = edge_index[0].astype(jnp.int32)
    dst = edge_index[1].astype(jnp.int32).reshape(NW, NCHUNK, C)
    batchb = jnp.broadcast_to(
        batch.astype(jnp.int32)[:, None], (N, BW))

    accp1, degp = _sc_agg_deg(x, src, dst)
    h1 = _tc_layer(x, accp1, degp, Wl1, bl1.reshape(1, D), Wr1)
    accp2 = _sc_agg(h1, src, dst)
    out = _tc_final(h1, accp2, degp, batchb, Wl2, bl2.reshape(1, D), Wr2,
                    Wout, bout.reshape(1, D))
    return out


# speedup vs baseline: 10.6577x; 1.0031x over previous
import functools

import jax
import jax.numpy as jnp
from jax import lax
from jax.experimental import pallas as pl
from jax.experimental.pallas import tpu as pltpu
from jax.experimental.pallas import tpu_sc as plsc

N = 10000
E = 320000
D = 128
G = 64

NC = 2
NS = 16
NW = NC * NS
EPW = E // NW
C = 80
NCHUNK = EPW // C
NBUF = 2
RPT = 624
RTAIL = N - NS * RPT


def _sc_agg_body(with_deg, h_hbm, src_hbm, dst_hbm, *refs):
    if with_deg:
        (acc_out, deg_out, acc_sh, sidx_v, didx_v, rows_v, zbuf,
         zsem, isem, gsem, ssem) = refs
    else:
        (acc_out, acc_sh, sidx_v, didx_v, rows_v, zbuf,
         zsem, isem, gsem, ssem) = refs

    cid = lax.axis_index("c")
    sid = lax.axis_index("s")
    wid = sid * NC + cid

    def zero_my_acc_slice():
        def za(j, _):
            pltpu.async_copy(zbuf,
                             acc_sh.at[pl.ds(sid * RPT + j * 16, 16)], zsem)
            return _

        lax.fori_loop(0, RPT // 16, za, 0)

        @pl.when(sid == NS - 1)
        def _():
            pltpu.async_copy(zbuf.at[pl.ds(0, RTAIL)],
                             acc_sh.at[pl.ds(NS * RPT, RTAIL)], zsem)

        def zw(j, _):
            pltpu.make_async_copy(
                zbuf, acc_sh.at[pl.ds(sid * RPT + j * 16, 16)], zsem).wait()
            return _

        lax.fori_loop(0, RPT // 16, zw, 0)

        @pl.when(sid == NS - 1)
        def _():
            pltpu.make_async_copy(
                zbuf.at[pl.ds(0, RTAIL)],
                acc_sh.at[pl.ds(NS * RPT, RTAIL)], zsem).wait()

    def copy_my_acc_slice(out_ref):
        pltpu.sync_copy(acc_sh.at[pl.ds(sid * RPT, RPT)],
                        out_ref.at[cid, pl.ds(sid * RPT, RPT)])

        @pl.when(sid == NS - 1)
        def _():
            pltpu.sync_copy(acc_sh.at[pl.ds(NS * RPT, RTAIL)],
                            out_ref.at[cid, pl.ds(NS * RPT, RTAIL)])

    zeros16 = jnp.zeros((16,), jnp.float32)

    def zb(i, _):
        zbuf[i // (D // 16), pl.ds((i % (D // 16)) * 16, 16)] = zeros16
        return _

    lax.fori_loop(0, 16 * (D // 16), zb, 0)

    zero_my_acc_slice()

    base = wid * EPW
    pltpu.sync_copy(dst_hbm.at[wid], didx_v)

    def start_sidx(g, b):
        pltpu.async_copy(src_hbm.at[pl.ds(base + g * C, C)], sidx_v[b],
                         isem[b])

    def wait_sidx(g, b):
        pltpu.make_async_copy(src_hbm.at[pl.ds(base + g * C, C)], sidx_v[b],
                              isem[b]).wait()

    def start_gather(g, b):
        return pltpu.async_copy(h_hbm.at[sidx_v[b]], rows_v[b], gsem[b])

    def wait_gather(g, b):
        pltpu.make_async_copy(h_hbm.at[sidx_v[b]], rows_v[b],
                              gsem[b]).wait()

    def scatter(g, b):
        pltpu.sync_copy(rows_v[b], acc_sh.at[didx_v.at[g]], add=True)

    def start_scatter(g, b):
        pltpu.async_copy(rows_v[b], acc_sh.at[didx_v.at[g]], ssem[b],
                         add=True)

    def wait_scatter(g, b):
        pltpu.make_async_copy(rows_v[b], acc_sh.at[didx_v.at[g]],
                              ssem[b]).wait()

    if with_deg:
        ones16 = jnp.ones((16,), jnp.float32)

        def ob(i, _):
            rows_v[0][i // (D // 16), pl.ds((i % (D // 16)) * 16, 16)] = ones16
            return _

        lax.fori_loop(0, C * (D // 16), ob, 0)

        plsc.subcore_barrier()

        def dgroup(j, carry):
            for k in range(5):
                pltpu.async_copy(rows_v[0], acc_sh.at[didx_v.at[j * 5 + k]],
                                 ssem[0], add=True)
            for k in range(5):
                pltpu.make_async_copy(rows_v[0],
                                      acc_sh.at[didx_v.at[j * 5 + k]],
                                      ssem[0]).wait()
            return carry

        lax.fori_loop(0, NCHUNK // 5, dgroup, 0)

        plsc.subcore_barrier()
        copy_my_acc_slice(deg_out)
        zero_my_acc_slice()

    plsc.subcore_barrier()

    start_sidx(0, 0)
    wait_sidx(0, 0)
    start_gather(0, 0)
    start_sidx(1, 1)

    def mpair(i, carry):
        g = i * 2

        @pl.when(g > 0)
        def _():
            wait_scatter(g - 1, 1)

        wait_sidx(g + 1, 1)
        start_gather(g + 1, 1)
        wait_gather(g, 0)

        @pl.when(g + 2 < NCHUNK)
        def _():
            start_sidx(g + 2, 0)

        start_scatter(g, 0)
        wait_scatter(g, 0)

        @pl.when(g + 2 < NCHUNK)
        def _():
            wait_sidx(g + 2, 0)
            start_gather(g + 2, 0)

        wait_gather(g + 1, 1)

        @pl.when(g + 3 < NCHUNK)
        def _():
            start_sidx(g + 3, 1)

        start_scatter(g + 1, 1)
        return carry

    lax.fori_loop(0, NCHUNK // 2, mpair, 0)

    wait_scatter(NCHUNK - 2, 1)
    wait_gather(NCHUNK - 1, 0)
    scatter(NCHUNK - 1, 0)

    plsc.subcore_barrier()
    copy_my_acc_slice(acc_out)


def _make_sc_agg(with_deg):
    mesh = plsc.VectorSubcoreMesh(core_axis_name="c", subcore_axis_name="s")
    out_type = [jax.ShapeDtypeStruct((NC, N, D), jnp.float32)]
    scratch = [
        pltpu.VMEM_SHARED((N, D), jnp.float32),
    ]
    if with_deg:
        out_type.append(jax.ShapeDtypeStruct((NC, N, D), jnp.float32))
    scratch += [
        [pltpu.VMEM((C,), jnp.int32) for _ in range(NBUF)],
        pltpu.VMEM((NCHUNK, C), jnp.int32),
        [pltpu.VMEM((C, D), jnp.float32) for _ in range(NBUF)],
        pltpu.VMEM((16, D), jnp.float32),
        pltpu.SemaphoreType.DMA,
        [pltpu.SemaphoreType.DMA for _ in range(NBUF)],
        [pltpu.SemaphoreType.DMA for _ in range(NBUF)],
        [pltpu.SemaphoreType.DMA for _ in range(NBUF)],
    ]

    return pl.kernel(
        functools.partial(_sc_agg_body, with_deg),
        out_type=tuple(out_type) if with_deg else out_type[0],
        mesh=mesh,
        scratch_types=scratch,
    )


_sc_agg_deg = _make_sc_agg(True)
_sc_agg = _make_sc_agg(False)


BN = 2000
BW = 16


def _tc_layer_body(x_ref, accp_ref, degp_ref, wl_ref, bl_ref, wr_ref, o_ref):
    acc = accp_ref[0] + accp_ref[1]
    dsum = degp_ref[0] + degp_ref[1]
    deg = jnp.maximum(dsum[:, 0:1], 1.0)
    mean = acc / deg
    h = (lax.dot_general(mean, wl_ref[...], (((1,), (1,)), ((), ())),
                         preferred_element_type=jnp.float32)
         + bl_ref[...]
         + lax.dot_general(x_ref[...], wr_ref[...], (((1,), (1,)), ((), ())),
                           preferred_element_type=jnp.float32))
    o_ref[...] = jnp.maximum(h, 0.0)


def _tc_layer(x, accp, degp, wl, bl, wr):
    grid = (N // BN,)
    return pl.pallas_call(
        _tc_layer_body,
        grid=grid,
        in_specs=[
            pl.BlockSpec((BN, D), lambda i: (i, 0)),
            pl.BlockSpec((NC, BN, D), lambda i: (0, i, 0)),
            pl.BlockSpec((NC, BN, D), lambda i: (0, i, 0)),
            pl.BlockSpec((D, D), lambda i: (0, 0)),
            pl.BlockSpec((1, D), lambda i: (0, 0)),
            pl.BlockSpec((D, D), lambda i: (0, 0)),
        ],
        out_specs=pl.BlockSpec((BN, D), lambda i: (i, 0)),
        out_shape=jax.ShapeDtypeStruct((N, D), jnp.float32),
    )(x, accp, degp, wl, bl, wr)


def _tc_final_body(h1_ref, accp_ref, degp_ref, batch_ref, wl_ref, bl_ref,
                   wr_ref, wout_ref, bout_ref, o_ref, ps_ref, pc_ref):
    i = pl.program_id(0)
    acc = accp_ref[0] + accp_ref[1]
    dsum = degp_ref[0] + degp_ref[1]
    deg = jnp.maximum(dsum[:, 0:1], 1.0)
    mean = acc / deg
    h = (lax.dot_general(mean, wl_ref[...], (((1,), (1,)), ((), ())),
                         preferred_element_type=jnp.float32)
         + bl_ref[...]
         + lax.dot_general(h1_ref[...], wr_ref[...], (((1,), (1,)), ((), ())),
                           preferred_element_type=jnp.float32))
    h2 = jnp.maximum(h, 0.0)

    bvals = batch_ref[:, 0:1]
    gids = lax.broadcasted_iota(jnp.int32, (BN, G), 1)
    oh = (bvals == gids).astype(jnp.float32)

    ps = lax.dot_general(oh, h2, (((0,), (0,)), ((), ())),
                         preferred_element_type=jnp.float32)
    ones_b = jnp.ones((BN, D), jnp.float32)
    pc = lax.dot_general(oh, ones_b, (((0,), (0,)), ((), ())),
                         preferred_element_type=jnp.float32)

    @pl.when(i == 0)
    def _():
        ps_ref[...] = ps
        pc_ref[...] = pc

    @pl.when(i != 0)
    def _():
        ps_ref[...] = ps_ref[...] + ps
        pc_ref[...] = pc_ref[...] + pc

    @pl.when(i == pl.num_programs(0) - 1)
    def _():
        pooled = ps_ref[...] / jnp.maximum(pc_ref[...], 1.0)
        o_ref[...] = (
            lax.dot_general(pooled, wout_ref[...], (((1,), (1,)), ((), ())),
                            preferred_element_type=jnp.float32)
            + bout_ref[...])


def _tc_final(h1, accp, degp, batchb, wl, bl, wr, wout, bout):
    grid = (N // BN,)
    return pl.pallas_call(
        _tc_final_body,
        grid=grid,
        in_specs=[
            pl.BlockSpec((BN, D), lambda i: (i, 0)),
            pl.BlockSpec((NC, BN, D), lambda i: (0, i, 0)),
            pl.BlockSpec((NC, BN, D), lambda i: (0, i, 0)),
            pl.BlockSpec((BN, BW), lambda i: (i, 0)),
            pl.BlockSpec((D, D), lambda i: (0, 0)),
            pl.BlockSpec((1, D), lambda i: (0, 0)),
            pl.BlockSpec((D, D), lambda i: (0, 0)),
            pl.BlockSpec((D, D), lambda i: (0, 0)),
            pl.BlockSpec((1, D), lambda i: (0, 0)),
        ],
        out_specs=pl.BlockSpec((G, D), lambda i: (0, 0)),
        out_shape=jax.ShapeDtypeStruct((G, D), jnp.float32),
        scratch_shapes=[
            pltpu.VMEM((G, D), jnp.float32),
            pltpu.VMEM((G, D), jnp.float32),
        ],
    )(h1, accp, degp, batchb, wl, bl, wr, wout, bout)


def kernel(x, edge_index, batch, Wl1, bl1, Wr1, Wl2, bl2, Wr2, Wout, bout):
    src = edge_index[0].astype(jnp.int32)
    dst = edge_index[1].astype(jnp.int32).reshape(NW, NCHUNK, C)
    batchb = jnp.broadcast_to(
        batch.astype(jnp.int32)[:, None], (N, BW))

    accp1, degp = _sc_agg_deg(x, src, dst)
    h1 = _tc_layer(x, accp1, degp, Wl1, bl1.reshape(1, D), Wr1)
    accp2 = _sc_agg(h1, src, dst)
    out = _tc_final(h1, accp2, degp, batchb, Wl2, bl2.reshape(1, D), Wr2,
                    Wout, bout.reshape(1, D))
    return out


# speedup vs baseline: 11.0847x; 1.0401x over previous
import functools

import jax
import jax.numpy as jnp
from jax import lax
from jax.experimental import pallas as pl
from jax.experimental.pallas import tpu as pltpu
from jax.experimental.pallas import tpu_sc as plsc

N = 10000
E = 320000
D = 128
G = 64

NC = 2
NS = 16
NW = NC * NS
EPW = E // NW
C = 80
NCHUNK = EPW // C
NBUF = 2
RPT = 624
RTAIL = N - NS * RPT
DEGW = 16


def _sc_agg_body(with_deg, h_hbm, src_hbm, dst_hbm, *refs):
    if with_deg:
        (acc_out, deg_out, acc_sh, sidx_v, didx_v, rows_v, zbuf,
         zsem, isem, gsem, ssem) = refs
    else:
        (acc_out, acc_sh, sidx_v, didx_v, rows_v, zbuf,
         zsem, isem, gsem, ssem) = refs

    cid = lax.axis_index("c")
    sid = lax.axis_index("s")
    wid = sid * NC + cid

    def zero_my_acc_slice():
        def za(j, _):
            pltpu.async_copy(zbuf,
                             acc_sh.at[pl.ds(sid * RPT + j * 16, 16)], zsem)
            return _

        lax.fori_loop(0, RPT // 16, za, 0)

        @pl.when(sid == NS - 1)
        def _():
            pltpu.async_copy(zbuf.at[pl.ds(0, RTAIL)],
                             acc_sh.at[pl.ds(NS * RPT, RTAIL)], zsem)

        def zw(j, _):
            pltpu.make_async_copy(
                zbuf, acc_sh.at[pl.ds(sid * RPT + j * 16, 16)], zsem).wait()
            return _

        lax.fori_loop(0, RPT // 16, zw, 0)

        @pl.when(sid == NS - 1)
        def _():
            pltpu.make_async_copy(
                zbuf.at[pl.ds(0, RTAIL)],
                acc_sh.at[pl.ds(NS * RPT, RTAIL)], zsem).wait()

    def copy_my_acc_slice(out_ref):
        pltpu.sync_copy(acc_sh.at[pl.ds(sid * RPT, RPT)],
                        out_ref.at[cid, pl.ds(sid * RPT, RPT)])

        @pl.when(sid == NS - 1)
        def _():
            pltpu.sync_copy(acc_sh.at[pl.ds(NS * RPT, RTAIL)],
                            out_ref.at[cid, pl.ds(NS * RPT, RTAIL)])

    zeros16 = jnp.zeros((16,), jnp.float32)

    def zb(i, _):
        zbuf[i // (D // 16), pl.ds((i % (D // 16)) * 16, 16)] = zeros16
        return _

    lax.fori_loop(0, 16 * (D // 16), zb, 0)

    zero_my_acc_slice()

    base = wid * EPW
    pltpu.sync_copy(dst_hbm.at[1, wid], didx_v)

    def start_sidx(g, b):
        pltpu.async_copy(src_hbm.at[pl.ds(base + g * C, C)], sidx_v[b],
                         isem[b])

    def wait_sidx(g, b):
        pltpu.make_async_copy(src_hbm.at[pl.ds(base + g * C, C)], sidx_v[b],
                              isem[b]).wait()

    def start_gather(g, b):
        return pltpu.async_copy(h_hbm.at[sidx_v[b]], rows_v[b], gsem[b])

    def wait_gather(g, b):
        pltpu.make_async_copy(h_hbm.at[sidx_v[b]], rows_v[b],
                              gsem[b]).wait()

    def scatter(g, b):
        pltpu.sync_copy(rows_v[b], acc_sh.at[didx_v.at[g]], add=True)

    def start_scatter(g, b):
        pltpu.async_copy(rows_v[b], acc_sh.at[didx_v.at[g]], ssem[b],
                         add=True)

    def wait_scatter(g, b):
        pltpu.make_async_copy(rows_v[b], acc_sh.at[didx_v.at[g]],
                              ssem[b]).wait()

    if with_deg:
        ones16 = jnp.ones((16,), jnp.float32)

        def ob(i, _):
            rows_v[0][i // (D // 16), pl.ds((i % (D // 16)) * 16, 16)] = ones16
            return _

        lax.fori_loop(0, C * (D // 16), ob, 0)

        plsc.subcore_barrier()

        def dgroup(j, carry):
            for k in range(5):
                pltpu.async_copy(rows_v[0], acc_sh.at[didx_v.at[j * 5 + k]],
                                 ssem[0], add=True)
            for k in range(5):
                pltpu.make_async_copy(rows_v[0],
                                      acc_sh.at[didx_v.at[j * 5 + k]],
                                      ssem[0]).wait()
            return carry

        lax.fori_loop(0, NCHUNK // 5, dgroup, 0)

        plsc.subcore_barrier()
        copy_my_acc_slice(deg_out)
        zero_my_acc_slice()

    plsc.subcore_barrier()

    start_sidx(0, 0)
    wait_sidx(0, 0)
    start_gather(0, 0)
    start_sidx(1, 1)

    def mpair(i, carry):
        g = i * 2

        @pl.when(g > 0)
        def _():
            wait_scatter(g - 1, 1)

        wait_sidx(g + 1, 1)
        start_gather(g + 1, 1)
        wait_gather(g, 0)

        @pl.when(g + 2 < NCHUNK)
        def _():
            start_sidx(g + 2, 0)

        start_scatter(g, 0)
        wait_scatter(g, 0)

        @pl.when(g + 2 < NCHUNK)
        def _():
            wait_sidx(g + 2, 0)
            start_gather(g + 2, 0)

        wait_gather(g + 1, 1)

        @pl.when(g + 3 < NCHUNK)
        def _():
            start_sidx(g + 3, 1)

        start_scatter(g + 1, 1)
        return carry

    lax.fori_loop(0, NCHUNK // 2, mpair, 0)

    wait_scatter(NCHUNK - 2, 1)
    wait_gather(NCHUNK - 1, 0)
    scatter(NCHUNK - 1, 0)

    plsc.subcore_barrier()
    copy_my_acc_slice(acc_out)


def _make_sc_agg(with_deg):
    mesh = plsc.VectorSubcoreMesh(core_axis_name="c", subcore_axis_name="s")
    out_type = [jax.ShapeDtypeStruct((NC, N, D), jnp.float32)]
    scratch = [
        pltpu.VMEM_SHARED((N, D), jnp.float32),
    ]
    if with_deg:
        out_type.append(jax.ShapeDtypeStruct((NC, N, D), jnp.float32))
    scratch += [
        [pltpu.VMEM((C,), jnp.int32) for _ in range(NBUF)],
        pltpu.VMEM((NCHUNK, C), jnp.int32),
        [pltpu.VMEM((C, D), jnp.float32) for _ in range(NBUF)],
        pltpu.VMEM((16, D), jnp.float32),
        pltpu.SemaphoreType.DMA,
        [pltpu.SemaphoreType.DMA for _ in range(NBUF)],
        [pltpu.SemaphoreType.DMA for _ in range(NBUF)],
        [pltpu.SemaphoreType.DMA for _ in range(NBUF)],
    ]

    return pl.kernel(
        functools.partial(_sc_agg_body, with_deg),
        out_type=tuple(out_type) if with_deg else out_type[0],
        mesh=mesh,
        scratch_types=scratch,
    )


_sc_agg_deg = _make_sc_agg(True)
_sc_agg = _make_sc_agg(False)


BN = 2000
BW = 16


def _tc_layer_body(x_ref, accp_ref, degp_ref, wl_ref, bl_ref, wr_ref, o_ref):
    acc = accp_ref[0] + accp_ref[1]
    dsum = degp_ref[0] + degp_ref[1]
    deg = jnp.maximum(dsum[:, 0:1], 1.0)
    mean = acc / deg
    h = (lax.dot_general(mean, wl_ref[...], (((1,), (1,)), ((), ())),
                         preferred_element_type=jnp.float32)
         + bl_ref[...]
         + lax.dot_general(x_ref[...], wr_ref[...], (((1,), (1,)), ((), ())),
                           preferred_element_type=jnp.float32))
    o_ref[...] = jnp.maximum(h, 0.0)


def _tc_layer(x, accp, degp, wl, bl, wr):
    grid = (N // BN,)
    return pl.pallas_call(
        _tc_layer_body,
        grid=grid,
        in_specs=[
            pl.BlockSpec((BN, D), lambda i: (i, 0)),
            pl.BlockSpec((NC, BN, D), lambda i: (0, i, 0)),
            pl.BlockSpec((NC, BN, DEGW), lambda i: (0, i, 0)),
            pl.BlockSpec((D, D), lambda i: (0, 0)),
            pl.BlockSpec((1, D), lambda i: (0, 0)),
            pl.BlockSpec((D, D), lambda i: (0, 0)),
        ],
        out_specs=pl.BlockSpec((BN, D), lambda i: (i, 0)),
        out_shape=jax.ShapeDtypeStruct((N, D), jnp.float32),
    )(x, accp, degp, wl, bl, wr)


def _tc_final_body(h1_ref, accp_ref, degp_ref, batch_ref, wl_ref, bl_ref,
                   wr_ref, wout_ref, bout_ref, o_ref, ps_ref, pc_ref):
    i = pl.program_id(0)
    acc = accp_ref[0] + accp_ref[1]
    dsum = degp_ref[0] + degp_ref[1]
    deg = jnp.maximum(dsum[:, 0:1], 1.0)
    mean = acc / deg
    h = (lax.dot_general(mean, wl_ref[...], (((1,), (1,)), ((), ())),
                         preferred_element_type=jnp.float32)
         + bl_ref[...]
         + lax.dot_general(h1_ref[...], wr_ref[...], (((1,), (1,)), ((), ())),
                           preferred_element_type=jnp.float32))
    h2 = jnp.maximum(h, 0.0)

    bvals = batch_ref[0]
    gids = lax.broadcasted_iota(jnp.int32, (G, BN), 0)
    oh = (bvals == gids).astype(jnp.float32)

    ps = lax.dot_general(oh, h2, (((1,), (0,)), ((), ())),
                         preferred_element_type=jnp.float32)
    ones_b = jnp.ones((BN, D), jnp.float32)
    pc = lax.dot_general(oh, ones_b, (((1,), (0,)), ((), ())),
                         preferred_element_type=jnp.float32)

    @pl.when(i == 0)
    def _():
        ps_ref[...] = ps
        pc_ref[...] = pc

    @pl.when(i != 0)
    def _():
        ps_ref[...] = ps_ref[...] + ps
        pc_ref[...] = pc_ref[...] + pc

    @pl.when(i == pl.num_programs(0) - 1)
    def _():
        pooled = ps_ref[...] / jnp.maximum(pc_ref[...], 1.0)
        o_ref[...] = (
            lax.dot_general(pooled, wout_ref[...], (((1,), (1,)), ((), ())),
                            preferred_element_type=jnp.float32)
            + bout_ref[...])


def _tc_final(h1, accp, degp, batchb, wl, bl, wr, wout, bout):
    grid = (N // BN,)
    return pl.pallas_call(
        _tc_final_body,
        grid=grid,
        in_specs=[
            pl.BlockSpec((BN, D), lambda i: (i, 0)),
            pl.BlockSpec((NC, BN, D), lambda i: (0, i, 0)),
            pl.BlockSpec((NC, BN, DEGW), lambda i: (0, i, 0)),
            pl.BlockSpec((1, 1, BN), lambda i: (i, 0, 0)),
            pl.BlockSpec((D, D), lambda i: (0, 0)),
            pl.BlockSpec((1, D), lambda i: (0, 0)),
            pl.BlockSpec((D, D), lambda i: (0, 0)),
            pl.BlockSpec((D, D), lambda i: (0, 0)),
            pl.BlockSpec((1, D), lambda i: (0, 0)),
        ],
        out_specs=pl.BlockSpec((G, D), lambda i: (0, 0)),
        out_shape=jax.ShapeDtypeStruct((G, D), jnp.float32),
        scratch_shapes=[
            pltpu.VMEM((G, D), jnp.float32),
            pltpu.VMEM((G, D), jnp.float32),
        ],
    )(h1, accp, degp, batchb, wl, bl, wr, wout, bout)


def kernel(x, edge_index, batch, Wl1, bl1, Wr1, Wl2, bl2, Wr2, Wout, bout):
    ei = edge_index.astype(jnp.int32)
    src = ei.reshape(2 * E)
    dst = ei.reshape(2, NW, NCHUNK, C)
    batchr = batch.astype(jnp.int32).reshape(N // BN, 1, BN)

    accp1, degp = _sc_agg_deg(x, src, dst)
    degp16 = degp[:, :, :DEGW]
    h1 = _tc_layer(x, accp1, degp16, Wl1, bl1.reshape(1, D), Wr1)
    accp2 = _sc_agg(h1, src, dst)
    out = _tc_final(h1, accp2, degp16, batchr, Wl2, bl2.reshape(1, D), Wr2,
                    Wout, bout.reshape(1, D))
    return out
